# Initial kernel scaffold; baseline (speedup 1.0000x reference)
import jax
import jax.numpy as jnp
from jax.experimental import pallas as pl


def kernel(t, pe):
    raise NotImplementedError("write your pallas kernel here")


# speedup vs baseline: 2.1879x; 2.1879x over previous
import functools

import jax
import jax.numpy as jnp
from jax import lax
from jax.experimental import pallas as pl
from jax.experimental.pallas import tpu as pltpu
from jax.experimental.pallas import tpu_sc as plsc

_SEQ_LENGTH = 8192
_D_MODEL = 1024
_BATCH = 4
_SEQ_LEN = 8192

_N_IDX = _BATCH * _SEQ_LEN
_NC, _NS = 2, 16
_NW = _NC * _NS
_PER_W = _N_IDX // _NW
_CHUNK = 64
_STEPS = _PER_W // _CHUNK


def _pe_lookup_body(t_hbm, pe_hbm, out_hbm, idx_v, rows_v, sem):
    wid = lax.axis_index("s") * _NC + lax.axis_index("c")
    base = wid * _PER_W
    pltpu.sync_copy(t_hbm.at[pl.ds(base, _PER_W)], idx_v)

    def step(c, _):
        off = c * _CHUNK
        pltpu.async_copy(
            pe_hbm.at[idx_v.at[pl.ds(off, _CHUNK)]], rows_v, sem
        ).wait()
        pltpu.sync_copy(rows_v, out_hbm.at[pl.ds(base + off, _CHUNK)])
        return _

    lax.fori_loop(0, _STEPS, step, None)


@jax.jit
def _pe_lookup(t_flat, pe):
    mesh = plsc.VectorSubcoreMesh(core_axis_name="c", subcore_axis_name="s")
    f = pl.kernel(
        _pe_lookup_body,
        out_type=jax.ShapeDtypeStruct((_N_IDX, _D_MODEL), jnp.float32),
        mesh=mesh,
        scratch_types=[
            pltpu.VMEM((_PER_W,), jnp.int32),
            pltpu.VMEM((_CHUNK, _D_MODEL), jnp.float32),
            pltpu.SemaphoreType.DMA,
        ],
    )
    return f(t_flat, pe)


def kernel(t, pe):
    out = _pe_lookup(t.reshape(-1), pe)
    return out.reshape(_BATCH, _SEQ_LEN, _D_MODEL)


# speedup vs baseline: 2.3650x; 1.0809x over previous
import functools

import jax
import jax.numpy as jnp
from jax import lax
from jax.experimental import pallas as pl
from jax.experimental.pallas import tpu as pltpu
from jax.experimental.pallas import tpu_sc as plsc

_SEQ_LENGTH = 8192
_D_MODEL = 1024
_BATCH = 4
_SEQ_LEN = 8192

_N_IDX = _BATCH * _SEQ_LEN
_NC, _NS = 2, 16
_NW = _NC * _NS
_PER_W = _N_IDX // _NW
_CHUNK = 32
_STEPS = _PER_W // _CHUNK


def _pe_lookup_body(t_hbm, pe_hbm, out_hbm, idx_v, rows0, rows1, sem0, sem1):
    wid = lax.axis_index("s") * _NC + lax.axis_index("c")
    base = wid * _PER_W
    rows = (rows0, rows1)
    sems = (sem0, sem1)
    pltpu.sync_copy(t_hbm.at[pl.ds(base, _PER_W)], idx_v)

    def start_gather(c, b):
        pltpu.async_copy(
            pe_hbm.at[idx_v.at[pl.ds(c * _CHUNK, _CHUNK)]], rows[b], sems[b]
        )

    start_gather(0, 0)

    def outer(k, _):
        for b in range(2):
            c = k * 2 + b

            @pl.when(c + 1 < _STEPS)
            def _start_next():
                start_gather(c + 1, 1 - b)

            pltpu.make_async_copy(
                pe_hbm.at[pl.ds(0, _CHUNK)], rows[b], sems[b]
            ).wait()
            pltpu.sync_copy(rows[b], out_hbm.at[pl.ds(base + c * _CHUNK, _CHUNK)])
        return _

    lax.fori_loop(0, _STEPS // 2, outer, None)


@jax.jit
def _pe_lookup(t_flat, pe):
    mesh = plsc.VectorSubcoreMesh(core_axis_name="c", subcore_axis_name="s")
    f = pl.kernel(
        _pe_lookup_body,
        out_type=jax.ShapeDtypeStruct((_N_IDX, _D_MODEL), jnp.float32),
        mesh=mesh,
        scratch_types=[
            pltpu.VMEM((_PER_W,), jnp.int32),
            pltpu.VMEM((_CHUNK, _D_MODEL), jnp.float32),
            pltpu.VMEM((_CHUNK, _D_MODEL), jnp.float32),
            pltpu.SemaphoreType.DMA,
            pltpu.SemaphoreType.DMA,
        ],
    )
    return f(t_flat, pe)


def kernel(t, pe):
    out = _pe_lookup(t.reshape(-1), pe)
    return out.reshape(_BATCH, _SEQ_LEN, _D_MODEL)


# speedup vs baseline: 2.3774x; 1.0053x over previous
import functools

import jax
import jax.numpy as jnp
from jax import lax
from jax.experimental import pallas as pl
from jax.experimental.pallas import tpu as pltpu
from jax.experimental.pallas import tpu_sc as plsc

_SEQ_LENGTH = 8192
_D_MODEL = 1024
_BATCH = 4
_SEQ_LEN = 8192

_N_IDX = _BATCH * _SEQ_LEN
_NC, _NS = 2, 16
_NW = _NC * _NS
_PER_W = _N_IDX // _NW
_CHUNK = 16
_STEPS = _PER_W // _CHUNK
_NBUF = 4


def _pe_lookup_body(t_hbm, pe_hbm, out_hbm, idx_v, *bufs):
    rows = bufs[:_NBUF]
    gsems = bufs[_NBUF:2 * _NBUF]
    ssems = bufs[2 * _NBUF:]
    wid = lax.axis_index("s") * _NC + lax.axis_index("c")
    base = wid * _PER_W
    pltpu.sync_copy(t_hbm.at[pl.ds(base, _PER_W)], idx_v)

    def start_gather(c, b):
        pltpu.async_copy(
            pe_hbm.at[idx_v.at[pl.ds(c * _CHUNK, _CHUNK)]], rows[b], gsems[b]
        )

    def start_store(c, b):
        pltpu.async_copy(rows[b], out_hbm.at[pl.ds(base + c * _CHUNK, _CHUNK)],
                         ssems[b])

    def wait_gather(b):
        pltpu.make_async_copy(pe_hbm.at[pl.ds(0, _CHUNK)], rows[b],
                              gsems[b]).wait()

    def wait_store(b):
        pltpu.make_async_copy(rows[b], out_hbm.at[pl.ds(base, _CHUNK)],
                              ssems[b]).wait()

    start_gather(0, 0)
    start_gather(1, 1)

    def outer(k, _):
        for b in range(_NBUF):
            c = k * _NBUF + b
            wait_gather(b)
            start_store(c, b)
            nxt = (b + 2) % _NBUF

            @pl.when(c >= 2)
            def _guard():
                wait_store(nxt)

            @pl.when(c + 2 < _STEPS)
            def _prefetch():
                start_gather(c + 2, nxt)
        return _

    lax.fori_loop(0, _STEPS // _NBUF, outer, None)
    wait_store((_STEPS - 2) % _NBUF)
    wait_store((_STEPS - 1) % _NBUF)


@jax.jit
def _pe_lookup(t_flat, pe):
    mesh = plsc.VectorSubcoreMesh(core_axis_name="c", subcore_axis_name="s")
    f = pl.kernel(
        _pe_lookup_body,
        out_type=jax.ShapeDtypeStruct((_N_IDX, _D_MODEL), jnp.float32),
        mesh=mesh,
        scratch_types=(
            [pltpu.VMEM((_PER_W,), jnp.int32)]
            + [pltpu.VMEM((_CHUNK, _D_MODEL), jnp.float32)] * _NBUF
            + [pltpu.SemaphoreType.DMA] * (2 * _NBUF)
        ),
    )
    return f(t_flat, pe)


def kernel(t, pe):
    out = _pe_lookup(t.reshape(-1), pe)
    return out.reshape(_BATCH, _SEQ_LEN, _D_MODEL)


# speedup vs baseline: 2.3929x; 1.0065x over previous
import functools

import jax
import jax.numpy as jnp
from jax import lax
from jax.experimental import pallas as pl
from jax.experimental.pallas import tpu as pltpu
from jax.experimental.pallas import tpu_sc as plsc

_SEQ_LENGTH = 8192
_D_MODEL = 1024
_BATCH = 4
_SEQ_LEN = 8192

_N_IDX = _BATCH * _SEQ_LEN
_NC, _NS = 2, 16
_NW = _NC * _NS
_PER_W = _N_IDX // _NW
_CHUNK = 8
_STEPS = _PER_W // _CHUNK
_NBUF = 8
_LEAD = 4


def _pe_lookup_body(t_hbm, pe_hbm, out_hbm, idx_v, *bufs):
    rows = bufs[:_NBUF]
    gsems = bufs[_NBUF:2 * _NBUF]
    ssems = bufs[2 * _NBUF:]
    wid = lax.axis_index("s") * _NC + lax.axis_index("c")
    base = wid * _PER_W
    pltpu.sync_copy(t_hbm.at[pl.ds(base, _PER_W)], idx_v)

    def start_gather(c, b):
        pltpu.async_copy(
            pe_hbm.at[idx_v.at[pl.ds(c * _CHUNK, _CHUNK)]], rows[b], gsems[b]
        )

    def start_store(c, b):
        pltpu.async_copy(rows[b], out_hbm.at[pl.ds(base + c * _CHUNK, _CHUNK)],
                         ssems[b])

    def wait_gather(b):
        pltpu.make_async_copy(pe_hbm.at[pl.ds(0, _CHUNK)], rows[b],
                              gsems[b]).wait()

    def wait_store(b):
        pltpu.make_async_copy(rows[b], out_hbm.at[pl.ds(base, _CHUNK)],
                              ssems[b]).wait()

    for j in range(_LEAD):
        start_gather(j, j)

    def outer(k, _):
        for b in range(_NBUF):
            c = k * _NBUF + b
            wait_gather(b)
            start_store(c, b)
            nxt = (b + _LEAD) % _NBUF

            @pl.when(c >= _NBUF - _LEAD)
            def _guard():
                wait_store(nxt)

            @pl.when(c + _LEAD < _STEPS)
            def _prefetch():
                start_gather(c + _LEAD, nxt)
        return _

    lax.fori_loop(0, _STEPS // _NBUF, outer, None)
    for j in range(_STEPS - (_NBUF - _LEAD), _STEPS):
        wait_store(j % _NBUF)


@jax.jit
def _pe_lookup(t_flat, pe):
    mesh = plsc.VectorSubcoreMesh(core_axis_name="c", subcore_axis_name="s")
    f = pl.kernel(
        _pe_lookup_body,
        out_type=jax.ShapeDtypeStruct((_N_IDX, _D_MODEL), jnp.float32),
        mesh=mesh,
        scratch_types=(
            [pltpu.VMEM((_PER_W,), jnp.int32)]
            + [pltpu.VMEM((_CHUNK, _D_MODEL), jnp.float32)] * _NBUF
            + [pltpu.SemaphoreType.DMA] * (2 * _NBUF)
        ),
    )
    return f(t_flat, pe)


def kernel(t, pe):
    out = _pe_lookup(t.reshape(-1), pe)
    return out.reshape(_BATCH, _SEQ_LEN, _D_MODEL)
